# Initial kernel scaffold; baseline (speedup 1.0000x reference)
import jax
import jax.numpy as jnp
from jax.experimental import pallas as pl


def kernel(x, edge_index, w, params):
    raise NotImplementedError("write your pallas kernel here")


# speedup vs baseline: 2.0294x; 2.0294x over previous
import functools

import jax
import jax.numpy as jnp
from jax import lax
from jax.experimental import pallas as pl
from jax.experimental.pallas import tpu as pltpu
from jax.experimental.pallas import tpu_sc as plsc

N = 10000
E = 320000
EMB = 128
NL = 5
BN_EPS = 1e-5
INV_SQRT = 1.0 / (1.0 + BN_EPS) ** 0.5

NC, NS, LANES = 2, 16, 16
NW = NC * NS
CHUNK = 128
CH_PER_TILE = 80
GRP = 8
NGRP = CH_PER_TILE // GRP
EPT = CHUNK * CH_PER_TILE
E_PAD = EPT * NW
N_ACC = 10240
STRIPE = N_ACC // NS

NK = 80
N_PAD = 10240
NCH_PER_TILE = 4


def _msg_body(h_hbm, ec_hbm, src_hbm, cat_hbm, dst_hbm, out_hbm,
              acc_sh, hrows, erows, src_t, cat_t, dst_t):
    c = lax.axis_index("c")
    s = lax.axis_index("s")
    w = c * NS + s

    @pl.loop(0, CHUNK)
    def _(e):
        for t in range(8):
            erows[e, pl.ds(t * 16, 16)] = jnp.zeros((16,), jnp.float32)

    for r in range(STRIPE // CHUNK):
        pltpu.sync_copy(erows, acc_sh.at[pl.ds(s * STRIPE + r * CHUNK, CHUNK)])
    plsc.subcore_barrier()

    @pl.loop(0, NGRP)
    def _(g):
        base = w * CH_PER_TILE + g * GRP
        pltpu.sync_copy(src_hbm.at[pl.ds(base, GRP)], src_t)
        pltpu.sync_copy(cat_hbm.at[pl.ds(base, GRP)], cat_t)
        pltpu.sync_copy(dst_hbm.at[pl.ds(base, GRP)], dst_t)

        @pl.loop(0, GRP)
        def _(k):
            pltpu.sync_copy(h_hbm.at[src_t.at[k]], hrows)
            pltpu.sync_copy(ec_hbm.at[cat_t.at[k]], erows)

            @pl.loop(0, CHUNK)
            def _(e):
                for t in range(8):
                    sl = pl.ds(t * 16, 16)
                    hrows[e, sl] = jnp.maximum(hrows[e, sl] + erows[e, sl], 0.0)

            pltpu.sync_copy(hrows, acc_sh.at[dst_t.at[k]], add=True)

    plsc.subcore_barrier()
    pltpu.sync_copy(acc_sh.at[pl.ds(s * STRIPE, STRIPE)],
                    out_hbm.at[c, pl.ds(s * STRIPE, STRIPE)])


_msg_kernel = functools.partial(
    pl.kernel,
    out_type=jax.ShapeDtypeStruct((NC, N_ACC, EMB), jnp.float32),
    mesh=plsc.VectorSubcoreMesh(core_axis_name="c", subcore_axis_name="s"),
    scratch_types=[
        pltpu.VMEM_SHARED((N_ACC, EMB), jnp.float32),
        pltpu.VMEM((CHUNK, EMB), jnp.float32),
        pltpu.VMEM((CHUNK, EMB), jnp.float32),
        pltpu.VMEM((GRP, CHUNK), jnp.int32),
        pltpu.VMEM((GRP, CHUNK), jnp.int32),
        pltpu.VMEM((GRP, CHUNK), jnp.int32),
    ],
)(_msg_body)


def _emb_body(t0_hbm, t1_hbm, x0_hbm, x1_hbm, out_hbm,
              arows, brows, x0_t, x1_t):
    c = lax.axis_index("c")
    s = lax.axis_index("s")
    w = c * NS + s
    pltpu.sync_copy(x0_hbm.at[pl.ds(w * NCH_PER_TILE, NCH_PER_TILE)], x0_t)
    pltpu.sync_copy(x1_hbm.at[pl.ds(w * NCH_PER_TILE, NCH_PER_TILE)], x1_t)

    @pl.loop(0, NCH_PER_TILE)
    def _(chunk):
        pltpu.sync_copy(t0_hbm.at[x0_t.at[chunk]], arows)
        pltpu.sync_copy(t1_hbm.at[x1_t.at[chunk]], brows)

        @pl.loop(0, NK)
        def _(e):
            for t in range(8):
                sl = pl.ds(t * 16, 16)
                arows[e, sl] = arows[e, sl] + brows[e, sl]

        pltpu.sync_copy(
            arows, out_hbm.at[pl.ds(w * (NCH_PER_TILE * NK) + chunk * NK, NK)])


_emb_kernel = functools.partial(
    pl.kernel,
    out_type=jax.ShapeDtypeStruct((N_PAD, EMB), jnp.float32),
    mesh=plsc.VectorSubcoreMesh(core_axis_name="c", subcore_axis_name="s"),
    scratch_types=[
        pltpu.VMEM((NK, EMB), jnp.float32),
        pltpu.VMEM((NK, EMB), jnp.float32),
        pltpu.VMEM((NCH_PER_TILE, NK), jnp.int32),
        pltpu.VMEM((NCH_PER_TILE, NK), jnp.int32),
    ],
)(_emb_body)


ROWS_BLK = 1000
GRID_N = N // ROWS_BLK


def _dense_body(relu_last, h_ref, vx_ref, a0_ref, a1_ref, eps_ref,
                w1_ref, b1_ref, g1_ref, bb1_ref, w2_ref, b2_ref,
                bng_ref, bnb_ref, hout_ref, cs_ref):
    hh = h_ref[...] + vx_ref[...]
    rst = (1.0 + eps_ref[...]) * hh + a0_ref[...] + a1_ref[...]
    z = jnp.dot(rst, w1_ref[...], preferred_element_type=jnp.float32) + b1_ref[...]
    z = z * (g1_ref[...] * INV_SQRT) + bb1_ref[...]
    z = jnp.maximum(z, 0.0)
    z = jnp.dot(z, w2_ref[...], preferred_element_type=jnp.float32) + b2_ref[...]
    z = z * (bng_ref[...] * INV_SQRT) + bnb_ref[...]
    if relu_last:
        z = jnp.maximum(z, 0.0)
    hout_ref[...] = z
    ps = jnp.sum(z, axis=0, keepdims=True)

    @pl.when(pl.program_id(0) == 0)
    def _():
        cs_ref[...] = ps

    @pl.when(pl.program_id(0) != 0)
    def _():
        cs_ref[...] += ps


def _make_dense(relu_last):
    row_spec = pl.BlockSpec((ROWS_BLK, EMB), lambda i: (i, 0))
    full = lambda shape: pl.BlockSpec(shape, lambda i: tuple(0 for _ in shape))
    return pl.pallas_call(
        functools.partial(_dense_body, relu_last),
        grid=(GRID_N,),
        in_specs=[
            row_spec,
            full((1, EMB)),
            row_spec,
            row_spec,
            full((1, 1)),
            full((EMB, 2 * EMB)),
            full((1, 2 * EMB)),
            full((1, 2 * EMB)),
            full((1, 2 * EMB)),
            full((2 * EMB, EMB)),
            full((1, EMB)),
            full((1, EMB)),
            full((1, EMB)),
        ],
        out_specs=[row_spec, full((1, EMB))],
        out_shape=[
            jax.ShapeDtypeStruct((N, EMB), jnp.float32),
            jax.ShapeDtypeStruct((1, EMB), jnp.float32),
        ],
    )


_dense_relu = _make_dense(True)
_dense_last = _make_dense(False)


def _vn_body(cs_ref, vx_ref, w1_ref, b1_ref, g1_ref, bb1_ref,
             w2_ref, b2_ref, g2_ref, bb2_ref, ectab_ref, vxo_ref, eco_ref):
    pooled = cs_ref[...] + vx_ref[...]
    t = jnp.dot(pooled, w1_ref[...], preferred_element_type=jnp.float32) + b1_ref[...]
    t = t * (g1_ref[...] * INV_SQRT) + bb1_ref[...]
    t = jnp.maximum(t, 0.0)
    t = jnp.dot(t, w2_ref[...], preferred_element_type=jnp.float32) + b2_ref[...]
    t = t * (g2_ref[...] * INV_SQRT) + bb2_ref[...]
    t = jnp.maximum(t, 0.0)
    vxn = vx_ref[...] + t
    vxo_ref[...] = vxn
    eco_ref[...] = ectab_ref[...] + vxn


_vn_kernel = pl.pallas_call(
    _vn_body,
    out_shape=[
        jax.ShapeDtypeStruct((1, EMB), jnp.float32),
        jax.ShapeDtypeStruct((256, EMB), jnp.float32),
    ],
)


def _prep_body(et0_ref, et1_ref, vx0_ref, ectabs_ref, ec0_ref):
    r = (et0_ref[...][:, None, :] + et1_ref[...][None, :, :]).reshape(1, 256, EMB)
    ectabs_ref[...] = r

    @pl.when(pl.program_id(0) == 0)
    def _():
        ec0_ref[...] = r[0] + vx0_ref[...]


_prep_kernel = pl.pallas_call(
    _prep_body,
    grid=(NL,),
    in_specs=[
        pl.BlockSpec((16, EMB), lambda i: (i, 0)),
        pl.BlockSpec((16, EMB), lambda i: (i, 0)),
        pl.BlockSpec((1, EMB), lambda i: (0, 0)),
    ],
    out_specs=[
        pl.BlockSpec((1, 256, EMB), lambda i: (i, 0, 0)),
        pl.BlockSpec((256, EMB), lambda i: (0, 0)),
    ],
    out_shape=[
        jax.ShapeDtypeStruct((NL, 256, EMB), jnp.float32),
        jax.ShapeDtypeStruct((256, EMB), jnp.float32),
    ],
)


def _readout_body(cs_ref, w_ref, b_ref, out_ref):
    s = jnp.sum(cs_ref[...] * w_ref[...]).reshape(1, 1)
    out_ref[...] = s * (1.0 / N) + b_ref[...]


_readout_kernel = pl.pallas_call(
    _readout_body,
    out_shape=jax.ShapeDtypeStruct((1, 1), jnp.float32),
)


def kernel(x, edge_index, w, params):
    p = params
    src = edge_index[0]
    dst = edge_index[1]
    cat = w[:, 0] * 16 + w[:, 1]

    pad = E_PAD - E
    srcp = jnp.concatenate([src, jnp.zeros((pad,), jnp.int32)]).reshape(
        NW * CH_PER_TILE, CHUNK)
    dstp = jnp.concatenate([dst, jnp.full((pad,), N, jnp.int32)]).reshape(
        NW * CH_PER_TILE, CHUNK)
    catp = jnp.concatenate([cat, jnp.zeros((pad,), jnp.int32)]).reshape(
        NW * CH_PER_TILE, CHUNK)

    npad = N_PAD - N
    x0p = jnp.concatenate([x[:, 0], jnp.zeros((npad,), jnp.int32)]).reshape(
        NW * NCH_PER_TILE, NK)
    x1p = jnp.concatenate([x[:, 1], jnp.zeros((npad,), jnp.int32)]).reshape(
        NW * NCH_PER_TILE, NK)

    h = _emb_kernel(p['nt0'], p['nt1'], x0p, x1p)[:N]

    et0s = jnp.concatenate([p['layers'][l]['et0'] for l in range(NL)], axis=0)
    et1s = jnp.concatenate([p['layers'][l]['et1'] for l in range(NL)], axis=0)
    vx = p['layers'][0]['vn_emb']
    ectabs, ec = _prep_kernel(et0s, et1s, vx)

    r2 = lambda a: a.reshape(1, -1)
    colsums = []
    for l in range(NL):
        lp = p['layers'][l]
        m = lp['gin_mlp']
        agg2 = _msg_kernel(h, ec, srcp, catp, dstp)
        dense = _dense_relu if l < NL - 1 else _dense_last
        h, cs = dense(
            h, vx, agg2[0], agg2[1], lp['gin_eps'].reshape(1, 1),
            m['W1'], r2(m['b1']), r2(m['g1']), r2(m['bb1']),
            m['W2'], r2(m['b2']), r2(lp['bn_g']), r2(lp['bn_b']))
        colsums.append(cs)
        if l < NL - 1:
            mv = lp['mlp_vn']
            vx, ec = _vn_kernel(
                cs, vx, mv['W1'], r2(mv['b1']), r2(mv['g1']), r2(mv['bb1']),
                mv['W2'], r2(mv['b2']), r2(mv['g2']), r2(mv['bb2']),
                ectabs[l + 1])

    cs_all = jnp.concatenate(colsums, axis=0)
    wout = p['Wout'][:, 0].reshape(NL, EMB)
    return _readout_kernel(cs_all, wout, p['bout'].reshape(1, 1))


# speedup vs baseline: 3.6511x; 1.7991x over previous
import functools

import jax
import jax.numpy as jnp
from jax import lax
from jax.experimental import pallas as pl
from jax.experimental.pallas import tpu as pltpu
from jax.experimental.pallas import tpu_sc as plsc

N = 10000
E = 320000
EMB = 128
NL = 5
BN_EPS = 1e-5
INV_SQRT = 1.0 / (1.0 + BN_EPS) ** 0.5

NC, NS, LANES = 2, 16, 16
NW = NC * NS
CHUNK = 64
CH_PER_TILE = 160
EPT = CHUNK * CH_PER_TILE
E_PAD = EPT * NW
N_ACC = 10240
STRIPE = N_ACC // NS
IPT = 3 * CH_PER_TILE

NK = 80
N_PAD = 10240
NCH_PER_TILE = 4


def _relu_add(hbuf, ebuf):
    @pl.loop(0, CHUNK)
    def _(e):
        for t in range(8):
            sl = pl.ds(t * 16, 16)
            hbuf[e, sl] = jnp.maximum(hbuf[e, sl] + ebuf[e, sl], 0.0)


def _msg_body(h_hbm, ec_hbm, idx_hbm, out_hbm,
              acc_sh, ha, hb, ea, eb, ia, ib, sha, she, shb, sheb):
    c = lax.axis_index("c")
    s = lax.axis_index("s")
    w = c * NS + s

    @pl.loop(0, CHUNK)
    def _(e):
        for t in range(8):
            ha[e, pl.ds(t * 16, 16)] = jnp.zeros((16,), jnp.float32)

    for r in range(STRIPE // CHUNK):
        pltpu.sync_copy(ha, acc_sh.at[pl.ds(s * STRIPE + r * CHUNK, CHUNK)])
    plsc.subcore_barrier()

    ibase = w * CH_PER_TILE

    pltpu.sync_copy(idx_hbm.at[ibase], ia)
    pltpu.async_copy(h_hbm.at[ia.at[0]], ha, sha)
    pltpu.async_copy(ec_hbm.at[ia.at[1]], ea, she)
    pltpu.sync_copy(idx_hbm.at[ibase + 1], ib)
    pltpu.async_copy(h_hbm.at[ib.at[0]], hb, shb)
    pltpu.async_copy(ec_hbm.at[ib.at[1]], eb, sheb)

    @pl.loop(0, CH_PER_TILE // 2)
    def _(j):
        pltpu.make_async_copy(h_hbm.at[ia.at[0]], ha, sha).wait()
        pltpu.make_async_copy(ec_hbm.at[ia.at[1]], ea, she).wait()
        _relu_add(ha, ea)
        pltpu.sync_copy(ha, acc_sh.at[ia.at[2]], add=True)
        pltpu.sync_copy(idx_hbm.at[ibase + 2 * j + 2], ia)
        pltpu.async_copy(h_hbm.at[ia.at[0]], ha, sha)
        pltpu.async_copy(ec_hbm.at[ia.at[1]], ea, she)
        pltpu.make_async_copy(h_hbm.at[ib.at[0]], hb, shb).wait()
        pltpu.make_async_copy(ec_hbm.at[ib.at[1]], eb, sheb).wait()
        _relu_add(hb, eb)
        pltpu.sync_copy(hb, acc_sh.at[ib.at[2]], add=True)
        pltpu.sync_copy(idx_hbm.at[ibase + 2 * j + 3], ib)
        pltpu.async_copy(h_hbm.at[ib.at[0]], hb, shb)
        pltpu.async_copy(ec_hbm.at[ib.at[1]], eb, sheb)

    pltpu.make_async_copy(h_hbm.at[ia.at[0]], ha, sha).wait()
    pltpu.make_async_copy(ec_hbm.at[ia.at[1]], ea, she).wait()
    pltpu.make_async_copy(h_hbm.at[ib.at[0]], hb, shb).wait()
    pltpu.make_async_copy(ec_hbm.at[ib.at[1]], eb, sheb).wait()

    plsc.subcore_barrier()
    pltpu.sync_copy(acc_sh.at[pl.ds(s * STRIPE, STRIPE)],
                    out_hbm.at[c, pl.ds(s * STRIPE, STRIPE)])


_msg_kernel = functools.partial(
    pl.kernel,
    out_type=jax.ShapeDtypeStruct((NC, N_ACC, EMB), jnp.float32),
    mesh=plsc.VectorSubcoreMesh(core_axis_name="c", subcore_axis_name="s"),
    scratch_types=[
        pltpu.VMEM_SHARED((N_ACC, EMB), jnp.float32),
        pltpu.VMEM((CHUNK, EMB), jnp.float32),
        pltpu.VMEM((CHUNK, EMB), jnp.float32),
        pltpu.VMEM((CHUNK, EMB), jnp.float32),
        pltpu.VMEM((CHUNK, EMB), jnp.float32),
        pltpu.VMEM((3, CHUNK), jnp.int32),
        pltpu.VMEM((3, CHUNK), jnp.int32),
        pltpu.SemaphoreType.DMA,
        pltpu.SemaphoreType.DMA,
        pltpu.SemaphoreType.DMA,
        pltpu.SemaphoreType.DMA,
    ],
)(_msg_body)


def _emb_body(t0_hbm, t1_hbm, x0_hbm, x1_hbm, out_hbm,
              arows, brows, x0_t, x1_t):
    c = lax.axis_index("c")
    s = lax.axis_index("s")
    w = c * NS + s
    pltpu.sync_copy(x0_hbm.at[pl.ds(w * NCH_PER_TILE, NCH_PER_TILE)], x0_t)
    pltpu.sync_copy(x1_hbm.at[pl.ds(w * NCH_PER_TILE, NCH_PER_TILE)], x1_t)

    @pl.loop(0, NCH_PER_TILE)
    def _(chunk):
        pltpu.sync_copy(t0_hbm.at[x0_t.at[chunk]], arows)
        pltpu.sync_copy(t1_hbm.at[x1_t.at[chunk]], brows)

        @pl.loop(0, NK)
        def _(e):
            for t in range(8):
                sl = pl.ds(t * 16, 16)
                arows[e, sl] = arows[e, sl] + brows[e, sl]

        pltpu.sync_copy(
            arows, out_hbm.at[pl.ds(w * (NCH_PER_TILE * NK) + chunk * NK, NK)])


_emb_kernel = functools.partial(
    pl.kernel,
    out_type=jax.ShapeDtypeStruct((N_PAD, EMB), jnp.float32),
    mesh=plsc.VectorSubcoreMesh(core_axis_name="c", subcore_axis_name="s"),
    scratch_types=[
        pltpu.VMEM((NK, EMB), jnp.float32),
        pltpu.VMEM((NK, EMB), jnp.float32),
        pltpu.VMEM((NCH_PER_TILE, NK), jnp.int32),
        pltpu.VMEM((NCH_PER_TILE, NK), jnp.int32),
    ],
)(_emb_body)


ROWS_BLK = 1000
GRID_N = N // ROWS_BLK


def _dense_body(relu_last, h_ref, vx_ref, a0_ref, a1_ref, eps_ref,
                w1_ref, b1_ref, g1_ref, bb1_ref, w2_ref, b2_ref,
                bng_ref, bnb_ref, hout_ref, cs_ref):
    hh = h_ref[...] + vx_ref[...]
    rst = (1.0 + eps_ref[...]) * hh + a0_ref[...] + a1_ref[...]
    z = jnp.dot(rst, w1_ref[...], preferred_element_type=jnp.float32) + b1_ref[...]
    z = z * (g1_ref[...] * INV_SQRT) + bb1_ref[...]
    z = jnp.maximum(z, 0.0)
    z = jnp.dot(z, w2_ref[...], preferred_element_type=jnp.float32) + b2_ref[...]
    z = z * (bng_ref[...] * INV_SQRT) + bnb_ref[...]
    if relu_last:
        z = jnp.maximum(z, 0.0)
    hout_ref[...] = z
    ps = jnp.sum(z, axis=0, keepdims=True)

    @pl.when(pl.program_id(0) == 0)
    def _():
        cs_ref[...] = ps

    @pl.when(pl.program_id(0) != 0)
    def _():
        cs_ref[...] += ps


def _make_dense(relu_last):
    row_spec = pl.BlockSpec((ROWS_BLK, EMB), lambda i: (i, 0))
    full = lambda shape: pl.BlockSpec(shape, lambda i: tuple(0 for _ in shape))
    return pl.pallas_call(
        functools.partial(_dense_body, relu_last),
        grid=(GRID_N,),
        in_specs=[
            row_spec,
            full((1, EMB)),
            row_spec,
            row_spec,
            full((1, 1)),
            full((EMB, 2 * EMB)),
            full((1, 2 * EMB)),
            full((1, 2 * EMB)),
            full((1, 2 * EMB)),
            full((2 * EMB, EMB)),
            full((1, EMB)),
            full((1, EMB)),
            full((1, EMB)),
        ],
        out_specs=[row_spec, full((1, EMB))],
        out_shape=[
            jax.ShapeDtypeStruct((N, EMB), jnp.float32),
            jax.ShapeDtypeStruct((1, EMB), jnp.float32),
        ],
    )


_dense_relu = _make_dense(True)
_dense_last = _make_dense(False)


def _vn_body(cs_ref, vx_ref, w1_ref, b1_ref, g1_ref, bb1_ref,
             w2_ref, b2_ref, g2_ref, bb2_ref, ectab_ref, vxo_ref, eco_ref):
    pooled = cs_ref[...] + vx_ref[...]
    t = jnp.dot(pooled, w1_ref[...], preferred_element_type=jnp.float32) + b1_ref[...]
    t = t * (g1_ref[...] * INV_SQRT) + bb1_ref[...]
    t = jnp.maximum(t, 0.0)
    t = jnp.dot(t, w2_ref[...], preferred_element_type=jnp.float32) + b2_ref[...]
    t = t * (g2_ref[...] * INV_SQRT) + bb2_ref[...]
    t = jnp.maximum(t, 0.0)
    vxn = vx_ref[...] + t
    vxo_ref[...] = vxn
    eco_ref[...] = ectab_ref[...] + vxn


_vn_kernel = pl.pallas_call(
    _vn_body,
    out_shape=[
        jax.ShapeDtypeStruct((1, EMB), jnp.float32),
        jax.ShapeDtypeStruct((256, EMB), jnp.float32),
    ],
)


def _prep_body(et0_ref, et1_ref, vx0_ref, ectabs_ref, ec0_ref):
    r = (et0_ref[...][:, None, :] + et1_ref[...][None, :, :]).reshape(1, 256, EMB)
    ectabs_ref[...] = r

    @pl.when(pl.program_id(0) == 0)
    def _():
        ec0_ref[...] = r[0] + vx0_ref[...]


_prep_kernel = pl.pallas_call(
    _prep_body,
    grid=(NL,),
    in_specs=[
        pl.BlockSpec((16, EMB), lambda i: (i, 0)),
        pl.BlockSpec((16, EMB), lambda i: (i, 0)),
        pl.BlockSpec((1, EMB), lambda i: (0, 0)),
    ],
    out_specs=[
        pl.BlockSpec((1, 256, EMB), lambda i: (i, 0, 0)),
        pl.BlockSpec((256, EMB), lambda i: (0, 0)),
    ],
    out_shape=[
        jax.ShapeDtypeStruct((NL, 256, EMB), jnp.float32),
        jax.ShapeDtypeStruct((256, EMB), jnp.float32),
    ],
)


def _readout_body(cs_ref, w_ref, b_ref, out_ref):
    s = jnp.sum(cs_ref[...] * w_ref[...]).reshape(1, 1)
    out_ref[...] = s * (1.0 / N) + b_ref[...]


_readout_kernel = pl.pallas_call(
    _readout_body,
    out_shape=jax.ShapeDtypeStruct((1, 1), jnp.float32),
)


def kernel(x, edge_index, w, params):
    p = params
    src = edge_index[0]
    dst = edge_index[1]
    cat = w[:, 0] * 16 + w[:, 1]

    pad = E_PAD - E
    srcp = jnp.concatenate([src, jnp.zeros((pad,), jnp.int32)]).reshape(
        NW, CH_PER_TILE, CHUNK)
    dstp = jnp.concatenate([dst, jnp.full((pad,), N, jnp.int32)]).reshape(
        NW, CH_PER_TILE, CHUNK)
    catp = jnp.concatenate([cat, jnp.zeros((pad,), jnp.int32)]).reshape(
        NW, CH_PER_TILE, CHUNK)
    idxpack = jnp.concatenate(
        [jnp.stack([srcp, catp, dstp], axis=2).reshape(
            NW * CH_PER_TILE, 3, CHUNK),
         jnp.zeros((2, 3, CHUNK), jnp.int32)], axis=0)

    npad = N_PAD - N
    x0p = jnp.concatenate([x[:, 0], jnp.zeros((npad,), jnp.int32)]).reshape(
        NW * NCH_PER_TILE, NK)
    x1p = jnp.concatenate([x[:, 1], jnp.zeros((npad,), jnp.int32)]).reshape(
        NW * NCH_PER_TILE, NK)

    h = _emb_kernel(p['nt0'], p['nt1'], x0p, x1p)[:N]

    et0s = jnp.concatenate([p['layers'][l]['et0'] for l in range(NL)], axis=0)
    et1s = jnp.concatenate([p['layers'][l]['et1'] for l in range(NL)], axis=0)
    vx = p['layers'][0]['vn_emb']
    ectabs, ec = _prep_kernel(et0s, et1s, vx)

    r2 = lambda a: a.reshape(1, -1)
    colsums = []
    for l in range(NL):
        lp = p['layers'][l]
        m = lp['gin_mlp']
        agg2 = _msg_kernel(h, ec, idxpack)
        dense = _dense_relu if l < NL - 1 else _dense_last
        h, cs = dense(
            h, vx, agg2[0], agg2[1], lp['gin_eps'].reshape(1, 1),
            m['W1'], r2(m['b1']), r2(m['g1']), r2(m['bb1']),
            m['W2'], r2(m['b2']), r2(lp['bn_g']), r2(lp['bn_b']))
        colsums.append(cs)
        if l < NL - 1:
            mv = lp['mlp_vn']
            vx, ec = _vn_kernel(
                cs, vx, mv['W1'], r2(mv['b1']), r2(mv['g1']), r2(mv['bb1']),
                mv['W2'], r2(mv['b2']), r2(mv['g2']), r2(mv['bb2']),
                ectabs[l + 1])

    cs_all = jnp.concatenate(colsums, axis=0)
    wout = p['Wout'][:, 0].reshape(NL, EMB)
    return _readout_kernel(cs_all, wout, p['bout'].reshape(1, 1))
